# baseline (speedup 1.0000x reference)
import functools

import jax
import jax.numpy as jnp
from jax.experimental import pallas as pl
from jax.experimental.pallas import tpu as pltpu

HIDDEN = 4096
NUM_EXPERTS = 64
TOP_K = 8
N_CHUNKS = 4


def _gate_kernel(x1_ref, x2_ref, w_ref, scores_ref, idx_ref):
    rows = x1_ref.shape[0]
    c = rows // N_CHUNKS
    h2 = x1_ref.shape[1]
    w1 = w_ref[:, :h2]
    w2 = w_ref[:, h2:]
    iota = jax.lax.broadcasted_iota(jnp.int32, (NUM_EXPERTS, c), 0)
    dn = (((1,), (1,)), ((), ()))
    logits = []
    for j in range(N_CHUNKS):
        l1 = jax.lax.dot_general(w1, x1_ref[j * c:(j + 1) * c, :], dn,
                                 preferred_element_type=jnp.float32)
        l2 = jax.lax.dot_general(w2, x2_ref[j * c:(j + 1) * c, :], dn,
                                 preferred_element_type=jnp.float32)
        logits.append(l1 + l2)
    for j in range(N_CHUNKS):
        l = logits[j]
        m = jnp.max(l, axis=0, keepdims=True)
        e = jnp.exp(l - m)
        p = e / jnp.sum(e, axis=0, keepdims=True)
        work = p
        vals = []
        idxs = []
        for _ in range(TOP_K):
            mk = jnp.max(work, axis=0, keepdims=True)
            hit = work == mk
            ik = jnp.min(jnp.where(hit, iota, NUM_EXPERTS), axis=0,
                         keepdims=True)
            vals.append(mk)
            idxs.append(ik)
            work = jnp.where(iota == ik, -1.0, work)
        top_p = jnp.concatenate(vals, axis=0)
        m2 = jnp.max(top_p, axis=0, keepdims=True)
        e2 = jnp.exp(top_p - m2)
        s = e2 / jnp.sum(e2, axis=0, keepdims=True)
        scores_ref[j * c:(j + 1) * c, :] = s.T
        idx_ref[j * c:(j + 1) * c, :] = jnp.concatenate(idxs, axis=0).T


@functools.partial(jax.jit, static_argnames=("block_rows",))
def _gate(x_flat, W, block_rows):
    rows = x_flat.shape[0]
    h2 = HIDDEN // 2
    grid = (rows // block_rows,)
    return pl.pallas_call(
        _gate_kernel,
        grid=grid,
        in_specs=[
            pl.BlockSpec((block_rows, h2), lambda i: (i, 0)),
            pl.BlockSpec((block_rows, h2), lambda i: (i, 1)),
            pl.BlockSpec((NUM_EXPERTS, HIDDEN), lambda i: (0, 0)),
        ],
        out_specs=[
            pl.BlockSpec((block_rows, TOP_K), lambda i: (i, 0)),
            pl.BlockSpec((block_rows, TOP_K), lambda i: (i, 0)),
        ],
        out_shape=[
            jax.ShapeDtypeStruct((rows, TOP_K), jnp.float32),
            jax.ShapeDtypeStruct((rows, TOP_K), jnp.int32),
        ],
        compiler_params=pltpu.CompilerParams(
            dimension_semantics=("parallel",),
        ),
    )(x_flat, x_flat, W)


def kernel(x, W):
    batch, seq, hidden = x.shape
    x_flat = x.reshape(batch * seq, hidden)
    top_scores, top_idx = _gate(x_flat, W, 512)
    aux = jnp.zeros((), dtype=x.dtype)
    return (top_scores, top_idx, aux)
